# baseline (speedup 1.0000x reference)
import jax
import jax.numpy as jnp
from jax.experimental import pallas as pl
from jax.experimental.pallas import tpu as pltpu

N = 10000
F_IN = 128
F_OUT = 128
BMC = 200
NC = N // BMC
NBUF = 4


def _gcn_kernel(x_ref, w_ref, adj_ref, bias_ref, out_ref,
                adj_buf, out_buf, support_ref, x_vmem,
                in_sem, out_sem, x_sem):
    def in_copy(i, slot):
        return pltpu.make_async_copy(
            adj_ref.at[pl.ds(i * BMC, BMC), :],
            adj_buf.at[slot],
            in_sem.at[slot],
        )

    def out_copy(i, slot):
        return pltpu.make_async_copy(
            out_buf.at[slot],
            out_ref.at[pl.ds(i * BMC, BMC), :],
            out_sem.at[slot],
        )

    for j in range(NBUF):
        in_copy(j, j).start()
    x_copy = pltpu.make_async_copy(x_ref, x_vmem, x_sem)
    x_copy.start()
    x_copy.wait()

    support_ref[...] = jnp.dot(
        x_vmem[...].astype(jnp.bfloat16),
        w_ref[...].astype(jnp.bfloat16),
        preferred_element_type=jnp.float32,
    ).astype(jnp.bfloat16)
    bias_v = bias_ref[...]

    def step(i, _):
        slot = jax.lax.rem(i, NBUF)
        in_copy(i, slot).wait()
        acc = jnp.dot(
            adj_buf[slot].astype(jnp.bfloat16),
            support_ref[...],
            preferred_element_type=jnp.float32,
        )

        @pl.when(i >= NBUF)
        def _():
            out_copy(i - NBUF, slot).wait()

        out_buf[slot] = acc + bias_v
        out_copy(i, slot).start()

        @pl.when(i + NBUF < NC)
        def _():
            in_copy(i + NBUF, slot).start()

        return 0

    jax.lax.fori_loop(0, NC, step, 0)

    for j in range(NBUF):
        i = NC - NBUF + j
        out_copy(i, i % NBUF).wait()


@jax.jit
def kernel(input, adj, weight, bias):
    return pl.pallas_call(
        _gcn_kernel,
        in_specs=[
            pl.BlockSpec(memory_space=pltpu.MemorySpace.HBM),
            pl.BlockSpec(memory_space=pltpu.VMEM),
            pl.BlockSpec(memory_space=pltpu.MemorySpace.HBM),
            pl.BlockSpec(memory_space=pltpu.VMEM),
        ],
        out_specs=pl.BlockSpec(memory_space=pltpu.MemorySpace.HBM),
        out_shape=jax.ShapeDtypeStruct((N, F_OUT), jnp.float32),
        scratch_shapes=[
            pltpu.VMEM((NBUF, BMC, N), jnp.float32),
            pltpu.VMEM((NBUF, BMC, F_OUT), jnp.float32),
            pltpu.VMEM((N, F_OUT), jnp.bfloat16),
            pltpu.VMEM((N, F_IN), jnp.float32),
            pltpu.SemaphoreType.DMA((NBUF,)),
            pltpu.SemaphoreType.DMA((NBUF,)),
            pltpu.SemaphoreType.DMA,
        ],
    )(input, weight, adj, bias.reshape(1, F_OUT))


# speedup vs baseline: 1.0225x; 1.0225x over previous
import jax
import jax.numpy as jnp
from jax.experimental import pallas as pl
from jax.experimental.pallas import tpu as pltpu

N = 10000
F_IN = 128
F_OUT = 128
BMC = 200
NC = N // BMC
NBUF = 4


def _gcn_kernel(x_ref, w_ref, adj_ref, bias_ref, out_ref,
                adj_buf, out_buf, support_ref, in_sem, out_sem):
    def in_copy(i, slot):
        return pltpu.make_async_copy(
            adj_ref.at[pl.ds(i * BMC, BMC), :],
            adj_buf.at[slot],
            in_sem.at[slot],
        )

    def out_copy(i, slot):
        return pltpu.make_async_copy(
            out_buf.at[slot],
            out_ref.at[pl.ds(i * BMC, BMC), :],
            out_sem.at[slot],
        )

    for j in range(NBUF):
        in_copy(j, j).start()

    support_ref[...] = jnp.dot(
        x_ref[...].astype(jnp.bfloat16),
        w_ref[...].astype(jnp.bfloat16),
        preferred_element_type=jnp.float32,
    ).astype(jnp.bfloat16)
    bias_v = bias_ref[...]

    def step(i, _):
        slot = jax.lax.rem(i, NBUF)
        in_copy(i, slot).wait()
        acc = jnp.dot(
            adj_buf[slot].astype(jnp.bfloat16),
            support_ref[...],
            preferred_element_type=jnp.float32,
        )

        @pl.when(i >= NBUF)
        def _():
            out_copy(i - NBUF, slot).wait()

        out_buf[slot] = acc + bias_v
        out_copy(i, slot).start()

        @pl.when(i + NBUF < NC)
        def _():
            in_copy(i + NBUF, slot).start()

        return 0

    jax.lax.fori_loop(0, NC, step, 0)

    for j in range(NBUF):
        i = NC - NBUF + j
        out_copy(i, i % NBUF).wait()


@jax.jit
def kernel(input, adj, weight, bias):
    return pl.pallas_call(
        _gcn_kernel,
        in_specs=[
            pl.BlockSpec(memory_space=pltpu.VMEM),
            pl.BlockSpec(memory_space=pltpu.VMEM),
            pl.BlockSpec(memory_space=pltpu.MemorySpace.HBM),
            pl.BlockSpec(memory_space=pltpu.VMEM),
        ],
        out_specs=pl.BlockSpec(memory_space=pltpu.MemorySpace.HBM),
        out_shape=jax.ShapeDtypeStruct((N, F_OUT), jnp.float32),
        scratch_shapes=[
            pltpu.VMEM((NBUF, BMC, N), jnp.float32),
            pltpu.VMEM((NBUF, BMC, F_OUT), jnp.float32),
            pltpu.VMEM((N, F_OUT), jnp.bfloat16),
            pltpu.SemaphoreType.DMA((NBUF,)),
            pltpu.SemaphoreType.DMA((NBUF,)),
        ],
    )(input, weight, adj, bias.reshape(1, F_OUT))


# speedup vs baseline: 1.0455x; 1.0226x over previous
import jax
import jax.numpy as jnp
from jax.experimental import pallas as pl
from jax.experimental.pallas import tpu as pltpu

N = 10000
F_IN = 128
F_OUT = 128
BM = 400
GRID = N // BM


def _gcn_kernel(x_ref, w_ref, adj_ref, bias_ref, out_ref, support_ref):
    @pl.when(pl.program_id(0) == 0)
    def _():
        support_ref[...] = jnp.dot(
            x_ref[...].astype(jnp.bfloat16),
            w_ref[...].astype(jnp.bfloat16),
            preferred_element_type=jnp.float32,
        ).astype(jnp.bfloat16)

    out_ref[...] = jnp.dot(
        adj_ref[...].astype(jnp.bfloat16),
        support_ref[...],
        preferred_element_type=jnp.float32,
    ) + bias_ref[...]


@jax.jit
def kernel(input, adj, weight, bias):
    return pl.pallas_call(
        _gcn_kernel,
        grid=(GRID,),
        in_specs=[
            pl.BlockSpec((N, F_IN), lambda i: (0, 0)),
            pl.BlockSpec((F_IN, F_OUT), lambda i: (0, 0)),
            pl.BlockSpec((BM, N), lambda i: (i, 0)),
            pl.BlockSpec((1, F_OUT), lambda i: (0, 0)),
        ],
        out_specs=pl.BlockSpec((BM, F_OUT), lambda i: (i, 0)),
        out_shape=jax.ShapeDtypeStruct((N, F_OUT), jnp.float32),
        scratch_shapes=[
            pltpu.VMEM((N, F_OUT), jnp.bfloat16),
        ],
    )(input, weight, adj, bias.reshape(1, F_OUT))
